# baseline (speedup 1.0000x reference)
import functools

import jax
import jax.numpy as jnp
from jax import lax
from jax.experimental import pallas as pl
from jax.experimental.pallas import tpu as pltpu
from jax.experimental.pallas import tpu_sc as plsc

H = 128
N_REG_N = 10000
N_URL_N = 50000
NPAD = 50176
NQ = NPAD // 4
RPT = NQ // 16
ZPT = (NQ + 16) // 16
BE = 128
RBLK = 1024
NBLK = NPAD // RBLK
DUMMY = NQ
EPAD_RU = 161792
EPAD_X = 200704

_mesh = plsc.VectorSubcoreMesh(core_axis_name="c", subcore_axis_name="s",
                               num_cores=2, num_subcores=16)


def _fill_rows(ref, nrows, width, val):
  v16 = jnp.full((16,), val, jnp.float32)
  for r in range(nrows):
    for g in range(width // 16):
      ref[r, pl.ds(g * 16, 16)] = v16


def _iota_idx(idx_v, base, nmax, lanes):
  for g in range(BE // 16):
    idx_v[pl.ds(g * 16, 16)] = jnp.minimum(base + g * 16 + lanes, nmax)


def _zero_indirect(acc, zrows, idx_v, s, nrows, lanes):
  per = -(-nrows // 16)
  nbat = -(-per // BE)
  base = s * per
  for b in range(nbat):
    _iota_idx(idx_v, base + b * BE, nrows - 1, lanes)
    pltpu.sync_copy(zrows, acc.at[idx_v])


def _readback(acc, rows_v, idx_v, out_h, wb, s, rpt, lanes, sem):
  base = s * rpt
  nfull = rpt // BE
  for b in range(nfull):
    _iota_idx(idx_v, base + b * BE, base + rpt - 1, lanes)
    pltpu.async_copy(acc.at[idx_v], rows_v, sem).wait()
    pltpu.sync_copy(rows_v, out_h.at[pl.ds(wb + b * BE, BE)])
  rem = rpt - nfull * BE
  if rem:
    _iota_idx(idx_v, base + nfull * BE, base + rpt - 1, lanes)
    pltpu.async_copy(acc.at[idx_v], rows_v, sem).wait()
    pltpu.sync_copy(rows_v.at[pl.ds(0, rem)],
                    out_h.at[pl.ds(wb + nfull * BE, rem)])


def _make_segsum(epad, n_src, ones_mode=False):
  epw = epad // 16
  nb = epw // BE
  scratch = [
      pltpu.VMEM_SHARED((NQ + 16, H), jnp.float32),
      pltpu.VMEM((BE,), jnp.int32),
      pltpu.VMEM((BE,), jnp.int32),
      pltpu.VMEM((BE,), jnp.int32),
      pltpu.VMEM((BE, H), jnp.float32),
      pltpu.SemaphoreType.DMA,
  ]
  if not ones_mode:
    scratch.insert(1, pltpu.VMEM((BE,), jnp.int32))

  def body(*args):
    if ones_mode:
      dst_h, out_h, acc, dst_v, idx_v, dl_v, rows_v, sem = args
    else:
      x_h, src_h, dst_h, out_h, acc, src_v, dst_v, idx_v, dl_v, rows_v, sem = args
    c = lax.axis_index("c")
    s = lax.axis_index("s")
    lanes = lax.iota(jnp.int32, 16)

    for p in range(2):
      lo = (2 * c + p) * NQ
      _fill_rows(rows_v, BE, H, 0.0)
      _zero_indirect(acc, rows_v, idx_v, s, NQ + 16, lanes)
      if ones_mode:
        _fill_rows(rows_v, BE, H, 1.0)
      plsc.subcore_barrier()

      e0 = s * epw

      def batch(i, _):
        eb = e0 + i * BE
        pltpu.sync_copy(dst_h.at[pl.ds(eb, BE)], dst_v)
        if not ones_mode:
          pltpu.sync_copy(src_h.at[pl.ds(eb, BE)], src_v)
        for g in range(BE // 16):
          sl = pl.ds(g * 16, 16)
          d16 = dst_v[sl]
          m = (d16 >= lo) & (d16 < lo + NQ)
          dl_v[sl] = jnp.where(m, d16 - lo, DUMMY)
          if not ones_mode:
            s16 = src_v[sl]
            idx_v[sl] = jnp.where(m, s16, 0)
        if not ones_mode:
          pltpu.async_copy(x_h.at[idx_v], rows_v, sem).wait()
        pltpu.sync_copy(rows_v, acc.at[dl_v], add=True)
        return 0

      lax.fori_loop(0, nb, batch, 0)
      plsc.subcore_barrier()
      _readback(acc, rows_v, idx_v, out_h, lo + s * RPT, s, RPT, lanes, sem)
      plsc.subcore_barrier()

  return pl.kernel(body, out_type=jax.ShapeDtypeStruct((NPAD, H), jnp.float32),
                   mesh=_mesh, scratch_types=scratch,
                   name=f"segsum_{epad}_{n_src}_{int(ones_mode)}")


_seg_ru = _make_segsum(EPAD_RU, N_REG_N)
_seg_x = _make_segsum(EPAD_X, NPAD)
_cnt_ru = _make_segsum(EPAD_RU, 0, ones_mode=True)
_cnt_x = _make_segsum(EPAD_X, 0, ones_mode=True)


def _dgt(a, w):
  return lax.dot_general(a, w, (((1,), (1,)), ((), ())),
                         preferred_element_type=jnp.float32)


def _prep_body(s_ref, cnt, wl, bsum, c_out):
  inv = 0.5 / jnp.maximum(cnt[:, 0:1], 1.0)
  c_out[...] = _dgt(s_ref[...] * inv, wl[...]) + 0.5 * bsum[...]


def _url_body(final, s_ref, cnt, c_ref, x, wlh, wra, wrb, wlin, blin, out):
  inv = 0.5 / jnp.maximum(cnt[:, 0:1], 1.0)
  t = _dgt(s_ref[...] * inv, wlh[...])
  u = _dgt(x[...], 0.5 * (wra[...] + wrb[...]))
  r = jnp.maximum(t + u + c_ref[...], 0.0)
  if final:
    out[...] = _dgt(r, wlin[...]) + blin[...]
  else:
    out[...] = r


def _har_body(s_ref, cnt, x, wl, wr, b, out):
  inv = 1.0 / jnp.maximum(cnt[:, 0:1], 1.0)
  t = _dgt(s_ref[...] * inv, wl[...])
  u = _dgt(x[...], wr[...])
  out[...] = jnp.maximum(t + u + b[...], 0.0)


def _full(shape):
  return pl.BlockSpec(shape, lambda i: (0,) * len(shape))


_ROW = pl.BlockSpec((RBLK, H), lambda i: (i, 0))
_WB = _full((H, H))
_BB = _full((1, H))
_OROW = jax.ShapeDtypeStruct((NPAD, H), jnp.float32)

_prep = pl.pallas_call(
    _prep_body, grid=(NBLK,),
    in_specs=[_ROW, _ROW, _WB, _BB],
    out_specs=_ROW, out_shape=_OROW)

_url_mid = pl.pallas_call(
    functools.partial(_url_body, False), grid=(NBLK,),
    in_specs=[_ROW, _ROW, _ROW, _ROW, _WB, _WB, _WB, _full((8, H)),
              _full((1, 8))],
    out_specs=_ROW, out_shape=_OROW)

_url_fin = pl.pallas_call(
    functools.partial(_url_body, True), grid=(NBLK,),
    in_specs=[_ROW, _ROW, _ROW, _ROW, _WB, _WB, _WB, _full((8, H)),
              _full((1, 8))],
    out_specs=pl.BlockSpec((RBLK, 8), lambda i: (i, 0)),
    out_shape=jax.ShapeDtypeStruct((NPAD, 8), jnp.float32))

_har_upd = pl.pallas_call(
    _har_body, grid=(NBLK,),
    in_specs=[_ROW, _ROW, _ROW, _WB, _WB, _BB],
    out_specs=_ROW, out_shape=_OROW)


def _pad_edges(ei, epad):
  e = ei.shape[1]
  src = jnp.concatenate([ei[0], jnp.zeros((epad - e,), jnp.int32)])
  dst = jnp.concatenate([ei[1], jnp.full((epad - e,), 1 << 30, jnp.int32)])
  return src, dst


def _pad_nodes(x):
  return jnp.concatenate(
      [x, jnp.zeros((NPAD - x.shape[0], H), jnp.float32)], axis=0)


def kernel(x_reg, x_url, x_har, ei_ru, ei_uh, ei_hu, Wl_ru, bl_ru, Wr_ru,
           Wl_uh, bl_uh, Wr_uh, Wl_hu, bl_hu, Wr_hu, W_lin, b_lin):
  xu = _pad_nodes(x_url)
  xh = _pad_nodes(x_har)
  s_ru, d_ru = _pad_edges(ei_ru, EPAD_RU)
  s_uh, d_uh = _pad_edges(ei_uh, EPAD_X)
  s_hu, d_hu = _pad_edges(ei_hu, EPAD_X)

  bsum = (bl_ru + bl_hu).reshape(1, H)
  bluh = bl_uh.reshape(1, H)
  wlin = jnp.zeros((8, H), jnp.float32).at[:2].set(W_lin)
  blin = jnp.zeros((1, 8), jnp.float32).at[0, :2].set(b_lin)

  cnt_ru = _cnt_ru(d_ru)
  cnt_hu = _cnt_x(d_hu)
  cnt_uh = _cnt_x(d_uh)

  sru = _seg_ru(x_reg, s_ru, d_ru)
  c_u = _prep(sru, cnt_ru, Wl_ru, bsum)

  shu = _seg_x(xh, s_hu, d_hu)
  suh = _seg_x(xu, s_uh, d_uh)

  for layer in range(3):
    if layer == 2:
      y = _url_fin(shu, cnt_hu, c_u, xu, Wl_hu, Wr_ru, Wr_hu, wlin, blin)
      return y[:N_URL_N, :2]
    nxu = _url_mid(shu, cnt_hu, c_u, xu, Wl_hu, Wr_ru, Wr_hu, wlin, blin)
    nxh = _har_upd(suh, cnt_uh, xh, Wl_uh, Wr_uh, bluh)
    xu, xh = nxu, nxh
    shu = _seg_x(xh, s_hu, d_hu)
    if layer == 0:
      suh = _seg_x(xu, s_uh, d_uh)


# speedup vs baseline: 13.5096x; 13.5096x over previous
import functools

import jax
import jax.numpy as jnp
from jax import lax
from jax.experimental import pallas as pl
from jax.experimental.pallas import tpu as pltpu
from jax.experimental.pallas import tpu_sc as plsc

H = 128
N_REG_N = 10000
N_URL_N = 50000
NPAD = 50176
NQ = NPAD // 4
RPT = NQ // 16
ZPT = (NQ + 16) // 16
BE = 128
RBLK = 1024
NBLK = NPAD // RBLK
DUMMY = NQ
EPAD_RU = 161792
EPAD_X = 200704

_mesh = plsc.VectorSubcoreMesh(core_axis_name="c", subcore_axis_name="s",
                               num_cores=2, num_subcores=16)


def _fill_rows(ref, nrows, width, val):
  v16 = jnp.full((16,), val, jnp.float32)
  for r in range(nrows):
    for g in range(width // 16):
      ref[r, pl.ds(g * 16, 16)] = v16


def _iota_idx(idx_v, base, nmax, lanes):
  for g in range(BE // 16):
    idx_v[pl.ds(g * 16, 16)] = jnp.minimum(base + g * 16 + lanes, nmax)


def _zero_indirect(acc, zrows, idx_v, s, nrows, lanes):
  per = -(-nrows // 16)
  nbat = -(-per // BE)
  base = s * per
  for b in range(nbat):
    _iota_idx(idx_v, base + b * BE, nrows - 1, lanes)
    pltpu.sync_copy(zrows, acc.at[idx_v])


def _readback(acc, rows_v, idx_v, out_h, wb, s, rpt, lanes, sem):
  base = s * rpt
  nfull = rpt // BE
  for b in range(nfull):
    _iota_idx(idx_v, base + b * BE, base + rpt - 1, lanes)
    pltpu.async_copy(acc.at[idx_v], rows_v, sem).wait()
    pltpu.sync_copy(rows_v, out_h.at[pl.ds(wb + b * BE, BE)])
  rem = rpt - nfull * BE
  if rem:
    _iota_idx(idx_v, base + nfull * BE, base + rpt - 1, lanes)
    pltpu.async_copy(acc.at[idx_v], rows_v, sem).wait()
    pltpu.sync_copy(rows_v.at[pl.ds(0, rem)],
                    out_h.at[pl.ds(wb + nfull * BE, rem)])


def _make_segsum(epad, n_src, ones_mode=False):
  epw = epad // 16
  nb = epw // BE
  scratch = [
      pltpu.VMEM_SHARED((NQ + 16, H), jnp.float32),
      pltpu.VMEM((BE,), jnp.int32),
      pltpu.VMEM((BE,), jnp.int32),
      pltpu.VMEM((BE,), jnp.int32),
      pltpu.VMEM((BE, H), jnp.float32),
      pltpu.SemaphoreType.DMA,
  ]
  if not ones_mode:
    scratch.insert(1, pltpu.VMEM((BE,), jnp.int32))

  def body(*args):
    if ones_mode:
      dst_h, out_h, acc, dst_v, idx_v, dl_v, rows_v, sem = args
    else:
      x_h, src_h, dst_h, out_h, acc, src_v, dst_v, idx_v, dl_v, rows_v, sem = args
    c = lax.axis_index("c")
    s = lax.axis_index("s")
    lanes = lax.iota(jnp.int32, 16)

    for p in range(2):
      lo = (2 * c + p) * NQ
      _fill_rows(rows_v, BE, H, 0.0)
      _zero_indirect(acc, rows_v, idx_v, s, NQ + 16, lanes)
      if ones_mode:
        _fill_rows(rows_v, BE, H, 1.0)
      plsc.subcore_barrier()

      e0 = s * epw

      def batch(i, _):
        eb = e0 + i * BE
        pltpu.sync_copy(dst_h.at[pl.ds(eb, BE)], dst_v)
        if not ones_mode:
          pltpu.sync_copy(src_h.at[pl.ds(eb, BE)], src_v)
        for g in range(BE // 16):
          sl = pl.ds(g * 16, 16)
          d16 = dst_v[sl]
          m = (d16 >= lo) & (d16 < lo + NQ)
          dl_v[sl] = jnp.where(m, d16 - lo, DUMMY + lanes)
          if not ones_mode:
            s16 = src_v[sl]
            idx_v[sl] = jnp.where(m, s16, lanes)
        if not ones_mode:
          pltpu.async_copy(x_h.at[idx_v], rows_v, sem).wait()
        pltpu.sync_copy(rows_v, acc.at[dl_v], add=True)
        return 0

      lax.fori_loop(0, nb, batch, 0)
      plsc.subcore_barrier()
      _readback(acc, rows_v, idx_v, out_h, lo + s * RPT, s, RPT, lanes, sem)
      plsc.subcore_barrier()

  return pl.kernel(body, out_type=jax.ShapeDtypeStruct((NPAD, H), jnp.float32),
                   mesh=_mesh, scratch_types=scratch,
                   name=f"segsum_{epad}_{n_src}_{int(ones_mode)}")


_seg_ru = _make_segsum(EPAD_RU, N_REG_N)
_seg_x = _make_segsum(EPAD_X, NPAD)
_cnt_ru = _make_segsum(EPAD_RU, 0, ones_mode=True)
_cnt_x = _make_segsum(EPAD_X, 0, ones_mode=True)


def _dgt(a, w):
  return lax.dot_general(a, w, (((1,), (1,)), ((), ())),
                         preferred_element_type=jnp.float32)


def _prep_body(s_ref, cnt, wl, bsum, c_out):
  inv = 0.5 / jnp.maximum(cnt[:, 0:1], 1.0)
  c_out[...] = _dgt(s_ref[...] * inv, wl[...]) + 0.5 * bsum[...]


def _url_body(final, s_ref, cnt, c_ref, x, wlh, wra, wrb, wlin, blin, out):
  inv = 0.5 / jnp.maximum(cnt[:, 0:1], 1.0)
  t = _dgt(s_ref[...] * inv, wlh[...])
  u = _dgt(x[...], 0.5 * (wra[...] + wrb[...]))
  r = jnp.maximum(t + u + c_ref[...], 0.0)
  if final:
    out[...] = _dgt(r, wlin[...]) + blin[...]
  else:
    out[...] = r


def _har_body(s_ref, cnt, x, wl, wr, b, out):
  inv = 1.0 / jnp.maximum(cnt[:, 0:1], 1.0)
  t = _dgt(s_ref[...] * inv, wl[...])
  u = _dgt(x[...], wr[...])
  out[...] = jnp.maximum(t + u + b[...], 0.0)


def _full(shape):
  return pl.BlockSpec(shape, lambda i: (0,) * len(shape))


_ROW = pl.BlockSpec((RBLK, H), lambda i: (i, 0))
_WB = _full((H, H))
_BB = _full((1, H))
_OROW = jax.ShapeDtypeStruct((NPAD, H), jnp.float32)

_prep = pl.pallas_call(
    _prep_body, grid=(NBLK,),
    in_specs=[_ROW, _ROW, _WB, _BB],
    out_specs=_ROW, out_shape=_OROW)

_url_mid = pl.pallas_call(
    functools.partial(_url_body, False), grid=(NBLK,),
    in_specs=[_ROW, _ROW, _ROW, _ROW, _WB, _WB, _WB, _full((8, H)),
              _full((1, 8))],
    out_specs=_ROW, out_shape=_OROW)

_url_fin = pl.pallas_call(
    functools.partial(_url_body, True), grid=(NBLK,),
    in_specs=[_ROW, _ROW, _ROW, _ROW, _WB, _WB, _WB, _full((8, H)),
              _full((1, 8))],
    out_specs=pl.BlockSpec((RBLK, 8), lambda i: (i, 0)),
    out_shape=jax.ShapeDtypeStruct((NPAD, 8), jnp.float32))

_har_upd = pl.pallas_call(
    _har_body, grid=(NBLK,),
    in_specs=[_ROW, _ROW, _ROW, _WB, _WB, _BB],
    out_specs=_ROW, out_shape=_OROW)


def _pad_edges(ei, epad):
  e = ei.shape[1]
  src = jnp.concatenate([ei[0], jnp.zeros((epad - e,), jnp.int32)])
  dst = jnp.concatenate([ei[1], jnp.full((epad - e,), 1 << 30, jnp.int32)])
  return src, dst


def _pad_nodes(x):
  return jnp.concatenate(
      [x, jnp.zeros((NPAD - x.shape[0], H), jnp.float32)], axis=0)


def kernel(x_reg, x_url, x_har, ei_ru, ei_uh, ei_hu, Wl_ru, bl_ru, Wr_ru,
           Wl_uh, bl_uh, Wr_uh, Wl_hu, bl_hu, Wr_hu, W_lin, b_lin):
  xu = _pad_nodes(x_url)
  xh = _pad_nodes(x_har)
  s_ru, d_ru = _pad_edges(ei_ru, EPAD_RU)
  s_uh, d_uh = _pad_edges(ei_uh, EPAD_X)
  s_hu, d_hu = _pad_edges(ei_hu, EPAD_X)

  bsum = (bl_ru + bl_hu).reshape(1, H)
  bluh = bl_uh.reshape(1, H)
  wlin = jnp.zeros((8, H), jnp.float32).at[:2].set(W_lin)
  blin = jnp.zeros((1, 8), jnp.float32).at[0, :2].set(b_lin)

  cnt_ru = _cnt_ru(d_ru)
  cnt_hu = _cnt_x(d_hu)
  cnt_uh = _cnt_x(d_uh)

  sru = _seg_ru(x_reg, s_ru, d_ru)
  c_u = _prep(sru, cnt_ru, Wl_ru, bsum)

  shu = _seg_x(xh, s_hu, d_hu)
  suh = _seg_x(xu, s_uh, d_uh)

  for layer in range(3):
    if layer == 2:
      y = _url_fin(shu, cnt_hu, c_u, xu, Wl_hu, Wr_ru, Wr_hu, wlin, blin)
      return y[:N_URL_N, :2]
    nxu = _url_mid(shu, cnt_hu, c_u, xu, Wl_hu, Wr_ru, Wr_hu, wlin, blin)
    nxh = _har_upd(suh, cnt_uh, xh, Wl_uh, Wr_uh, bluh)
    xu, xh = nxu, nxh
    shu = _seg_x(xh, s_hu, d_hu)
    if layer == 0:
      suh = _seg_x(xu, s_uh, d_uh)


# speedup vs baseline: 13.7749x; 1.0196x over previous
import functools

import jax
import jax.numpy as jnp
from jax import lax
from jax.experimental import pallas as pl
from jax.experimental.pallas import tpu as pltpu
from jax.experimental.pallas import tpu_sc as plsc

H = 128
N_REG_N = 10000
N_URL_N = 50000
NPAD = 50176
NQ = NPAD // 4
RPT = NQ // 16
ZPT = (NQ + 16) // 16
BE = 128
RBLK = 1024
NBLK = NPAD // RBLK
DUMMY = NQ
EPAD_RU = 163840
EPAD_X = 200704

_mesh = plsc.VectorSubcoreMesh(core_axis_name="c", subcore_axis_name="s",
                               num_cores=2, num_subcores=16)


def _fill_rows(ref, nrows, width, val):
  v16 = jnp.full((16,), val, jnp.float32)
  for r in range(nrows):
    for g in range(width // 16):
      ref[r, pl.ds(g * 16, 16)] = v16


def _iota_idx(idx_v, base, nmax, lanes):
  for g in range(BE // 16):
    idx_v[pl.ds(g * 16, 16)] = jnp.minimum(base + g * 16 + lanes, nmax)


def _zero_indirect(acc, zrows, idx_v, s, nrows, lanes):
  per = -(-nrows // 16)
  nbat = -(-per // BE)
  base = s * per
  for b in range(nbat):
    _iota_idx(idx_v, base + b * BE, nrows - 1, lanes)
    pltpu.sync_copy(zrows, acc.at[idx_v])


def _readback(acc, rows_v, idx_v, out_h, wb, s, rpt, lanes, sem):
  base = s * rpt
  nfull = rpt // BE
  for b in range(nfull):
    _iota_idx(idx_v, base + b * BE, base + rpt - 1, lanes)
    pltpu.async_copy(acc.at[idx_v], rows_v, sem).wait()
    pltpu.sync_copy(rows_v, out_h.at[pl.ds(wb + b * BE, BE)])
  rem = rpt - nfull * BE
  if rem:
    _iota_idx(idx_v, base + nfull * BE, base + rpt - 1, lanes)
    pltpu.async_copy(acc.at[idx_v], rows_v, sem).wait()
    pltpu.sync_copy(rows_v.at[pl.ds(0, rem)],
                    out_h.at[pl.ds(wb + nfull * BE, rem)])


def _make_segsum(epad, n_src, ones_mode=False):
  epw = epad // 16
  nb = epw // BE
  grp = 8 if nb % 8 == 0 else 7
  assert nb % grp == 0
  scratch = [
      pltpu.VMEM_SHARED((NQ + 16, H), jnp.float32),
      pltpu.VMEM((grp * BE,), jnp.int32),
      pltpu.VMEM((BE,), jnp.int32),
      pltpu.VMEM((BE,), jnp.int32),
      pltpu.VMEM((BE, H), jnp.float32),
      pltpu.SemaphoreType.DMA,
  ]
  if not ones_mode:
    scratch.insert(1, pltpu.VMEM((grp * BE,), jnp.int32))

  def body(*args):
    if ones_mode:
      dst_h, out_h, acc, dst_big, idx_v, dl_v, rows_v, sem = args
    else:
      x_h, src_h, dst_h, out_h, acc, src_big, dst_big, idx_v, dl_v, rows_v, sem = args
    c = lax.axis_index("c")
    s = lax.axis_index("s")
    lanes = lax.iota(jnp.int32, 16)

    for p in range(2):
      lo = (2 * c + p) * NQ
      _fill_rows(rows_v, BE, H, 0.0)
      _zero_indirect(acc, rows_v, idx_v, s, NQ + 16, lanes)
      if ones_mode:
        _fill_rows(rows_v, BE, H, 1.0)
      plsc.subcore_barrier()

      e0 = s * epw

      def group(i, _):
        eb = e0 + i * (grp * BE)
        pltpu.sync_copy(dst_h.at[pl.ds(eb, grp * BE)], dst_big)
        if not ones_mode:
          pltpu.sync_copy(src_h.at[pl.ds(eb, grp * BE)], src_big)
        for b in range(grp):
          for g in range(BE // 16):
            sl = pl.ds(b * BE + g * 16, 16)
            slo = pl.ds(g * 16, 16)
            d16 = dst_big[sl]
            m = (d16 >= lo) & (d16 < lo + NQ)
            dl_v[slo] = jnp.where(m, d16 - lo, DUMMY + lanes)
            if not ones_mode:
              s16 = src_big[sl]
              idx_v[slo] = jnp.where(m, s16, lanes)
          if not ones_mode:
            pltpu.async_copy(x_h.at[idx_v], rows_v, sem).wait()
          pltpu.sync_copy(rows_v, acc.at[dl_v], add=True)
        return 0

      lax.fori_loop(0, nb // grp, group, 0)
      plsc.subcore_barrier()
      _readback(acc, rows_v, idx_v, out_h, lo + s * RPT, s, RPT, lanes, sem)
      plsc.subcore_barrier()

  return pl.kernel(body, out_type=jax.ShapeDtypeStruct((NPAD, H), jnp.float32),
                   mesh=_mesh, scratch_types=scratch,
                   name=f"segsum_{epad}_{n_src}_{int(ones_mode)}")


_seg_ru = _make_segsum(EPAD_RU, N_REG_N)
_seg_x = _make_segsum(EPAD_X, NPAD)
_cnt_ru = _make_segsum(EPAD_RU, 0, ones_mode=True)
_cnt_x = _make_segsum(EPAD_X, 0, ones_mode=True)


def _dgt(a, w):
  return lax.dot_general(a, w, (((1,), (1,)), ((), ())),
                         preferred_element_type=jnp.float32)


def _prep_body(s_ref, cnt, wl, bsum, c_out):
  inv = 0.5 / jnp.maximum(cnt[:, 0:1], 1.0)
  c_out[...] = _dgt(s_ref[...] * inv, wl[...]) + 0.5 * bsum[...]


def _url_body(final, s_ref, cnt, c_ref, x, wlh, wra, wrb, wlin, blin, out):
  inv = 0.5 / jnp.maximum(cnt[:, 0:1], 1.0)
  t = _dgt(s_ref[...] * inv, wlh[...])
  u = _dgt(x[...], 0.5 * (wra[...] + wrb[...]))
  r = jnp.maximum(t + u + c_ref[...], 0.0)
  if final:
    out[...] = _dgt(r, wlin[...]) + blin[...]
  else:
    out[...] = r


def _har_body(s_ref, cnt, x, wl, wr, b, out):
  inv = 1.0 / jnp.maximum(cnt[:, 0:1], 1.0)
  t = _dgt(s_ref[...] * inv, wl[...])
  u = _dgt(x[...], wr[...])
  out[...] = jnp.maximum(t + u + b[...], 0.0)


def _full(shape):
  return pl.BlockSpec(shape, lambda i: (0,) * len(shape))


_ROW = pl.BlockSpec((RBLK, H), lambda i: (i, 0))
_WB = _full((H, H))
_BB = _full((1, H))
_OROW = jax.ShapeDtypeStruct((NPAD, H), jnp.float32)

_prep = pl.pallas_call(
    _prep_body, grid=(NBLK,),
    in_specs=[_ROW, _ROW, _WB, _BB],
    out_specs=_ROW, out_shape=_OROW)

_url_mid = pl.pallas_call(
    functools.partial(_url_body, False), grid=(NBLK,),
    in_specs=[_ROW, _ROW, _ROW, _ROW, _WB, _WB, _WB, _full((8, H)),
              _full((1, 8))],
    out_specs=_ROW, out_shape=_OROW)

_url_fin = pl.pallas_call(
    functools.partial(_url_body, True), grid=(NBLK,),
    in_specs=[_ROW, _ROW, _ROW, _ROW, _WB, _WB, _WB, _full((8, H)),
              _full((1, 8))],
    out_specs=pl.BlockSpec((RBLK, 8), lambda i: (i, 0)),
    out_shape=jax.ShapeDtypeStruct((NPAD, 8), jnp.float32))

_har_upd = pl.pallas_call(
    _har_body, grid=(NBLK,),
    in_specs=[_ROW, _ROW, _ROW, _WB, _WB, _BB],
    out_specs=_ROW, out_shape=_OROW)


def _pad_edges(ei, epad):
  e = ei.shape[1]
  src = jnp.concatenate([ei[0], jnp.zeros((epad - e,), jnp.int32)])
  dst = jnp.concatenate([ei[1], jnp.full((epad - e,), 1 << 30, jnp.int32)])
  return src, dst


def _pad_nodes(x):
  return jnp.concatenate(
      [x, jnp.zeros((NPAD - x.shape[0], H), jnp.float32)], axis=0)


def kernel(x_reg, x_url, x_har, ei_ru, ei_uh, ei_hu, Wl_ru, bl_ru, Wr_ru,
           Wl_uh, bl_uh, Wr_uh, Wl_hu, bl_hu, Wr_hu, W_lin, b_lin):
  xu = _pad_nodes(x_url)
  xh = _pad_nodes(x_har)
  s_ru, d_ru = _pad_edges(ei_ru, EPAD_RU)
  s_uh, d_uh = _pad_edges(ei_uh, EPAD_X)
  s_hu, d_hu = _pad_edges(ei_hu, EPAD_X)

  bsum = (bl_ru + bl_hu).reshape(1, H)
  bluh = bl_uh.reshape(1, H)
  wlin = jnp.zeros((8, H), jnp.float32).at[:2].set(W_lin)
  blin = jnp.zeros((1, 8), jnp.float32).at[0, :2].set(b_lin)

  cnt_ru = _cnt_ru(d_ru)
  cnt_hu = _cnt_x(d_hu)
  cnt_uh = _cnt_x(d_uh)

  sru = _seg_ru(x_reg, s_ru, d_ru)
  c_u = _prep(sru, cnt_ru, Wl_ru, bsum)

  shu = _seg_x(xh, s_hu, d_hu)
  suh = _seg_x(xu, s_uh, d_uh)

  for layer in range(3):
    if layer == 2:
      y = _url_fin(shu, cnt_hu, c_u, xu, Wl_hu, Wr_ru, Wr_hu, wlin, blin)
      return y[:N_URL_N, :2]
    nxu = _url_mid(shu, cnt_hu, c_u, xu, Wl_hu, Wr_ru, Wr_hu, wlin, blin)
    nxh = _har_upd(suh, cnt_uh, xh, Wl_uh, Wr_uh, bluh)
    xu, xh = nxu, nxh
    shu = _seg_x(xh, s_hu, d_hu)
    if layer == 0:
      suh = _seg_x(xu, s_uh, d_uh)
